# baseline (speedup 1.0000x reference)
import jax
import jax.numpy as jnp
from jax.experimental import pallas as pl
from jax.experimental.pallas import tpu as pltpu

B, S, N, IN, OUT, H = 2, 2, 256, 128, 64, 4
HID = OUT
BS = B * S
BI = 8
NB = N // BI
FC = 8
NEG = 0.2


def _body(h_ref, adjb_ref, wl_ref, wr_ref, attn_ref, out_ref,
          gl_s, grt_s, li_s, rj_s, e_s):
    hd = pl.program_id(1)

    h = h_ref[0]
    wl = wl_ref[0]
    wr = wr_ref[0]
    attn = attn_ref[...]

    gl = jnp.dot(h, wl, preferred_element_type=jnp.float32)
    gr = jnp.dot(h, wr, preferred_element_type=jnp.float32)

    li = jnp.dot(gl, attn, preferred_element_type=jnp.float32)
    rj = jnp.dot(gr, attn, preferred_element_type=jnp.float32)

    gl_s[...] = gl.reshape(NB, BI, HID)
    grt_s[...] = gr.T
    li_s[...] = (li * (NEG / FC)).reshape(NB, BI, 1)
    rj_s[...] = (rj * (NEG / FC)).T

    c8 = attn * (1.0 - NEG)

    def eblock(i0):
        glb = gl_s[i0]
        rank1 = ((li_s[i0] + rj_s[...])
                 + adjb_ref[pl.ds(i0 * BI, BI), :])
        acc = jnp.broadcast_to(rank1[:, None, :], (BI, FC, N))
        for fc in range(HID // FC):
            t = (glb[:, fc * FC:(fc + 1) * FC, None]
                 + grt_s[fc * FC:(fc + 1) * FC, :][None, :, :])
            acc = acc + jnp.maximum(t, 0.0) * c8[None, fc * FC:(fc + 1) * FC, :]
        return jnp.sum(acc, axis=1)

    for u in range(NB):
        e_s[pl.ds(u * BI, BI), :] = eblock(u)

    p = jnp.exp(e_s[...])
    s = jnp.sum(p, axis=1, keepdims=True)
    o = (jnp.dot(p, gr, preferred_element_type=jnp.float32)
         * ((1.0 / H) / s))

    @pl.when(hd == 0)
    def _init():
        out_ref[0] = o

    @pl.when(hd != 0)
    def _acc():
        out_ref[0] = out_ref[0] + o


@jax.jit
def kernel(h, adj, W_l, W_r, attn):
    h4 = h.reshape(BS, N, IN)
    adjb = jnp.where(adj == 0, jnp.float32(-1e9 / FC), jnp.float32(0.0))
    wl4 = W_l.reshape(IN, H, HID).swapaxes(0, 1)
    wr4 = W_r.reshape(IN, H, HID).swapaxes(0, 1)
    attn2 = attn.reshape(HID, 1)

    out = pl.pallas_call(
        _body,
        grid=(BS, H),
        in_specs=[
            pl.BlockSpec((1, N, IN), lambda b, hd: (b, 0, 0)),
            pl.BlockSpec((N, N), lambda b, hd: (0, 0)),
            pl.BlockSpec((1, IN, HID), lambda b, hd: (hd, 0, 0)),
            pl.BlockSpec((1, IN, HID), lambda b, hd: (hd, 0, 0)),
            pl.BlockSpec((HID, 1), lambda b, hd: (0, 0)),
        ],
        out_specs=pl.BlockSpec((1, N, OUT), lambda b, hd: (b, 0, 0)),
        out_shape=jax.ShapeDtypeStruct((BS, N, OUT), jnp.float32),
        scratch_shapes=[
            pltpu.VMEM((NB, BI, HID), jnp.float32),
            pltpu.VMEM((HID, N), jnp.float32),
            pltpu.VMEM((NB, BI, 1), jnp.float32),
            pltpu.VMEM((1, N), jnp.float32),
            pltpu.VMEM((N, N), jnp.float32),
        ],
    )(h4, adjb, wl4, wr4, attn2)
    return out.reshape(B, S, N, OUT)


# speedup vs baseline: 1.0157x; 1.0157x over previous
import jax
import jax.numpy as jnp
from jax.experimental import pallas as pl
from jax.experimental.pallas import tpu as pltpu

B, S, N, IN, OUT, H = 2, 2, 256, 128, 64, 4
HID = OUT
BS = B * S
BI = 8
NB = N // BI
FC = 8
NEG = 0.2


def _body(h_ref, adjb_ref, wl_ref, wr_ref, attn_ref, out_ref,
          gl_s, grt_s, li_s, rj_s, e_s):
    hd = pl.program_id(1)

    h = h_ref[0]
    wl = wl_ref[0]
    wr = wr_ref[0]
    attn = attn_ref[...]

    gl = jnp.dot(h, wl, preferred_element_type=jnp.float32)
    gr = jnp.dot(h, wr, preferred_element_type=jnp.float32)
    grt = jax.lax.dot_general(wr, h, (((0,), (1,)), ((), ())),
                              preferred_element_type=jnp.float32)

    li = jnp.dot(gl, attn, preferred_element_type=jnp.float32)
    rj = jnp.dot(gr, attn, preferred_element_type=jnp.float32)

    gl_s[...] = gl.reshape(NB, BI, HID)
    grt_s[...] = grt
    li_s[...] = (li * (NEG / FC)).reshape(NB, BI, 1)
    rj_s[...] = (rj * (NEG / FC)).T

    c8 = attn * (1.0 - NEG)

    def eblock(i0):
        glb = gl_s[i0]
        rank1 = ((li_s[i0] + rj_s[...])
                 + adjb_ref[pl.ds(i0 * BI, BI), :])
        acc = jnp.broadcast_to(rank1[:, None, :], (BI, FC, N))
        for fc in range(HID // FC):
            t = (glb[:, fc * FC:(fc + 1) * FC, None]
                 + grt_s[fc * FC:(fc + 1) * FC, :][None, :, :])
            acc = acc + jnp.maximum(t, 0.0) * c8[None, fc * FC:(fc + 1) * FC, :]
        return jnp.sum(acc, axis=1)

    for u in range(NB):
        e_s[pl.ds(u * BI, BI), :] = eblock(u)

    p = jnp.exp(e_s[...])
    s = jnp.sum(p, axis=1, keepdims=True)
    o = (jnp.dot(p, gr, preferred_element_type=jnp.float32)
         * ((1.0 / H) / s))

    @pl.when(hd == 0)
    def _init():
        out_ref[0] = o

    @pl.when(hd != 0)
    def _acc():
        out_ref[0] = out_ref[0] + o


@jax.jit
def kernel(h, adj, W_l, W_r, attn):
    h4 = h.reshape(BS, N, IN)
    adjb = jnp.where(adj == 0, jnp.float32(-1e9 / FC), jnp.float32(0.0))
    wl4 = W_l.reshape(IN, H, HID).swapaxes(0, 1)
    wr4 = W_r.reshape(IN, H, HID).swapaxes(0, 1)
    attn2 = attn.reshape(HID, 1)

    out = pl.pallas_call(
        _body,
        grid=(BS, H),
        in_specs=[
            pl.BlockSpec((1, N, IN), lambda b, hd: (b, 0, 0)),
            pl.BlockSpec((N, N), lambda b, hd: (0, 0)),
            pl.BlockSpec((1, IN, HID), lambda b, hd: (hd, 0, 0)),
            pl.BlockSpec((1, IN, HID), lambda b, hd: (hd, 0, 0)),
            pl.BlockSpec((HID, 1), lambda b, hd: (0, 0)),
        ],
        out_specs=pl.BlockSpec((1, N, OUT), lambda b, hd: (b, 0, 0)),
        out_shape=jax.ShapeDtypeStruct((BS, N, OUT), jnp.float32),
        scratch_shapes=[
            pltpu.VMEM((NB, BI, HID), jnp.float32),
            pltpu.VMEM((HID, N), jnp.float32),
            pltpu.VMEM((NB, BI, 1), jnp.float32),
            pltpu.VMEM((1, N), jnp.float32),
            pltpu.VMEM((N, N), jnp.float32),
        ],
    )(h4, adjb, wl4, wr4, attn2)
    return out.reshape(B, S, N, OUT)


# speedup vs baseline: 1.0471x; 1.0309x over previous
import jax
import jax.numpy as jnp
from jax.experimental import pallas as pl
from jax.experimental.pallas import tpu as pltpu

B, S, N, IN, OUT, H = 2, 2, 256, 128, 64, 4
HID = OUT
BS = B * S
BI = 8
NB = N // BI
FC = 8
NEG = 0.2


def _body(h_ref, adjb_ref, wl_ref, wr_ref, attn_ref, out_ref,
          gl_s, grt_s, li_s, rj_s, e_s):
    hd = pl.program_id(1)

    h = h_ref[0]
    wl = wl_ref[0]
    wr = wr_ref[0]
    attn = attn_ref[...]

    gl = jnp.dot(h, wl, preferred_element_type=jnp.float32)
    gr = jnp.dot(h, wr, preferred_element_type=jnp.float32)
    grt = jax.lax.dot_general(wr, h, (((0,), (1,)), ((), ())),
                              preferred_element_type=jnp.float32)

    li = jnp.dot(gl, attn, preferred_element_type=jnp.float32)
    rj = jax.lax.dot_general(attn, grt, (((0,), (0,)), ((), ())),
                             preferred_element_type=jnp.float32)

    gl_s[...] = gl.reshape(NB, BI, HID)
    grt_s[...] = grt
    li_s[...] = (li * (NEG / FC)).reshape(NB, BI, 1)
    rj_s[...] = rj * (NEG / FC)

    c8 = attn * (1.0 - NEG)

    def eblock(i0):
        glb = gl_s[i0]
        rank1 = ((li_s[i0] + rj_s[...])
                 + adjb_ref[pl.ds(i0 * BI, BI), :])
        acc = jnp.broadcast_to(rank1[:, None, :], (BI, FC, N))
        for fc in range(HID // FC):
            t = (glb[:, fc * FC:(fc + 1) * FC, None]
                 + grt_s[fc * FC:(fc + 1) * FC, :][None, :, :])
            acc = acc + jnp.maximum(t, 0.0) * c8[None, fc * FC:(fc + 1) * FC, :]
        return jnp.sum(acc, axis=1)

    for u in range(NB):
        e_s[pl.ds(u * BI, BI), :] = eblock(u)

    p = jnp.exp(e_s[...])
    s = jnp.sum(p, axis=1, keepdims=True)
    o = (jnp.dot(p, gr, preferred_element_type=jnp.float32)
         * ((1.0 / H) / s))

    @pl.when(hd == 0)
    def _init():
        out_ref[0] = o

    @pl.when(hd != 0)
    def _acc():
        out_ref[0] = out_ref[0] + o


@jax.jit
def kernel(h, adj, W_l, W_r, attn):
    h4 = h.reshape(BS, N, IN)
    adjb = jnp.where(adj == 0, jnp.float32(-1e9 / FC), jnp.float32(0.0))
    wl4 = W_l.reshape(IN, H, HID).swapaxes(0, 1)
    wr4 = W_r.reshape(IN, H, HID).swapaxes(0, 1)
    attn2 = attn.reshape(HID, 1)

    out = pl.pallas_call(
        _body,
        grid=(BS, H),
        in_specs=[
            pl.BlockSpec((1, N, IN), lambda b, hd: (b, 0, 0)),
            pl.BlockSpec((N, N), lambda b, hd: (0, 0)),
            pl.BlockSpec((1, IN, HID), lambda b, hd: (hd, 0, 0)),
            pl.BlockSpec((1, IN, HID), lambda b, hd: (hd, 0, 0)),
            pl.BlockSpec((HID, 1), lambda b, hd: (0, 0)),
        ],
        out_specs=pl.BlockSpec((1, N, OUT), lambda b, hd: (b, 0, 0)),
        out_shape=jax.ShapeDtypeStruct((BS, N, OUT), jnp.float32),
        scratch_shapes=[
            pltpu.VMEM((NB, BI, HID), jnp.float32),
            pltpu.VMEM((HID, N), jnp.float32),
            pltpu.VMEM((NB, BI, 1), jnp.float32),
            pltpu.VMEM((1, N), jnp.float32),
            pltpu.VMEM((N, N), jnp.float32),
        ],
    )(h4, adjb, wl4, wr4, attn2)
    return out.reshape(B, S, N, OUT)
